# Initial kernel scaffold; baseline (speedup 1.0000x reference)
import jax
import jax.numpy as jnp
from jax.experimental import pallas as pl


def kernel(x, edge, fc1_Wl, fc1_Wr, fc1_att, fc1_bias, fc1_bl, fc1_br, fc3_W, fc3_b, gc1_Wl, gc1_Wr, gc1_att, gc1_bias, gc1_bl, gc1_br, gc2_Wl, gc2_Wr, gc2_att, gc2_bias, gc2_bl, gc2_br, gru0_Whh, gru0_Wih, gru0_bhh, gru0_bih, gru1_Whh, gru1_Wih, gru1_bhh, gru1_bih, rc1_Wl, rc1_Wr, rc1_att, rc1_bias, rc1_bl, rc1_br, rc2_W, rc2_b, rc3_W, rc3_b):
    raise NotImplementedError("write your pallas kernel here")


# speedup vs baseline: 180.9210x; 180.9210x over previous
import functools

import jax
import jax.numpy as jnp
from jax.experimental import pallas as pl

_HEADS = 2
_NEG = -1e30


def _dot(a, b, dims):
    return jax.lax.dot_general(
        a, b, (dims, ((), ())),
        preferred_element_type=jnp.float32,
        precision=jax.lax.Precision.HIGHEST)


def _gat_body(x_full_ref, x_blk_ref, m_ref, wl_ref, bl_ref, wr_ref, br_ref,
              att_ref, bias_ref, o_ref, *, n, c, bi, bj):
    hc = _HEADS * c
    xt = x_full_ref[0]
    xb = x_blk_ref[0]
    xl = _dot(xt, wl_ref[...], ((1,), (1,))) + bl_ref[...]
    xr = _dot(xb, wr_ref[...], ((1,), (1,))) + br_ref[...]
    att = att_ref[...][0]

    ib = pl.program_id(1)
    col = jax.lax.broadcasted_iota(jnp.int32, (bi, n), 1)
    row = jax.lax.broadcasted_iota(jnp.int32, (bi, n), 0) + ib * bi
    mf = m_ref[0].astype(jnp.float32)
    diag = jnp.where(col == row, 1.0, 0.0)
    neg = jnp.where(mf + diag > 0, 0.0, _NEG)

    tiles = []
    for jt in range(n // bj):
        xlt = xl[jt * bj:(jt + 1) * bj]
        t3 = xr[:, None, :] + xlt[None, :, :]
        t3 = jnp.where(t3 >= 0, t3, 0.2 * t3)
        w3 = t3 * att[None, None, :]
        tiles.append((jnp.sum(w3[:, :, :c], axis=-1),
                      jnp.sum(w3[:, :, c:], axis=-1)))

    acc = None
    for h in range(_HEADS):
        alpha = jnp.concatenate([p[h] for p in tiles], axis=1) + neg
        amax = jnp.max(alpha, axis=1, keepdims=True)
        ex = jnp.exp(alpha - amax)
        den = jnp.sum(ex, axis=1, keepdims=True) + 1e-16
        a = ex / den
        outh = _dot(a, xl[:, h * c:(h + 1) * c], ((1,), (0,)))
        acc = outh if acc is None else acc + outh
    res = acc * (1.0 / _HEADS) + bias_ref[...]
    o_ref[0] = jnp.where(res > 0, res, jnp.exp(jnp.minimum(res, 0.0)) - 1.0)


def _gat(x, mask_t, wl, bl, wr, br, att, bias):
    t_, n, in_ch = x.shape
    hc = wl.shape[0]
    c = hc // _HEADS
    bi = 128
    bj = 64
    body = functools.partial(_gat_body, n=n, c=c, bi=bi, bj=bj)
    return pl.pallas_call(
        body,
        grid=(t_, n // bi),
        in_specs=[
            pl.BlockSpec((1, n, in_ch), lambda t, i: (t, 0, 0)),
            pl.BlockSpec((1, bi, in_ch), lambda t, i: (t, i, 0)),
            pl.BlockSpec((1, bi, n), lambda t, i: (t, i, 0)),
            pl.BlockSpec((hc, in_ch), lambda t, i: (0, 0)),
            pl.BlockSpec((1, hc), lambda t, i: (0, 0)),
            pl.BlockSpec((hc, in_ch), lambda t, i: (0, 0)),
            pl.BlockSpec((1, hc), lambda t, i: (0, 0)),
            pl.BlockSpec((1, hc), lambda t, i: (0, 0)),
            pl.BlockSpec((1, c), lambda t, i: (0, 0)),
        ],
        out_specs=pl.BlockSpec((1, bi, c), lambda t, i: (t, i, 0)),
        out_shape=jax.ShapeDtypeStruct((t_, n, c), jnp.float32),
    )(x, x, mask_t, wl, bl.reshape(1, hc), wr, br.reshape(1, hc),
      att.reshape(1, hc), bias.reshape(1, c))


def _gru_body(e_ref, wih0_ref, whh0_ref, bih0_ref, bhh0_ref,
              wih1_ref, whh1_ref, bih1_ref, bhh1_ref, o_ref, *, t_, bn, hd):
    def step(xt, h, wih, whh, bih, bhh):
        gi = _dot(xt, wih, ((1,), (1,))) + bih
        gh = _dot(h, whh, ((1,), (1,))) + bhh
        r = 1.0 / (1.0 + jnp.exp(-(gi[:, :hd] + gh[:, :hd])))
        z = 1.0 / (1.0 + jnp.exp(-(gi[:, hd:2 * hd] + gh[:, hd:2 * hd])))
        nc = jnp.tanh(gi[:, 2 * hd:] + r * gh[:, 2 * hd:])
        return (1.0 - z) * nc + z * h

    h = jnp.zeros((bn, hd), jnp.float32)
    ys = []
    for t in range(t_):
        h = step(e_ref[t], h, wih0_ref[...], whh0_ref[...],
                 bih0_ref[...], bhh0_ref[...])
        ys.append(h)
    h = jnp.zeros((bn, hd), jnp.float32)
    for t in range(t_):
        h = step(ys[t], h, wih1_ref[...], whh1_ref[...],
                 bih1_ref[...], bhh1_ref[...])
        o_ref[t] = h


def _gru2(e, wih0, whh0, bih0, bhh0, wih1, whh1, bih1, bhh1):
    t_, n, hd = e.shape
    g3 = 3 * hd
    bn = 256
    body = functools.partial(_gru_body, t_=t_, bn=bn, hd=hd)
    wspec = pl.BlockSpec((g3, hd), lambda i: (0, 0))
    bspec = pl.BlockSpec((1, g3), lambda i: (0, 0))
    return pl.pallas_call(
        body,
        grid=(n // bn,),
        in_specs=[pl.BlockSpec((t_, bn, hd), lambda i: (0, i, 0)),
                  wspec, wspec, bspec, bspec, wspec, wspec, bspec, bspec],
        out_specs=pl.BlockSpec((t_, bn, hd), lambda i: (0, i, 0)),
        out_shape=jax.ShapeDtypeStruct((t_, n, hd), jnp.float32),
    )(e, wih0, whh0, bih0.reshape(1, g3), bhh0.reshape(1, g3),
      wih1, whh1, bih1.reshape(1, g3), bhh1.reshape(1, g3))


def _head_body(g_ref, w2_ref, b2_ref, w3_ref, b3_ref, o_ref):
    r = jnp.tanh(_dot(g_ref[...], w2_ref[...], ((1,), (1,))) + b2_ref[...])
    o_ref[...] = _dot(r, w3_ref[...], ((1,), (1,))) + b3_ref[...]


def _head(g2d, w2, b2, w3, b3):
    rows, c_in = g2d.shape
    c_mid = w2.shape[0]
    c_out = w3.shape[0]
    bn = 256
    return pl.pallas_call(
        _head_body,
        grid=(rows // bn,),
        in_specs=[pl.BlockSpec((bn, c_in), lambda i: (i, 0)),
                  pl.BlockSpec((c_mid, c_in), lambda i: (0, 0)),
                  pl.BlockSpec((1, c_mid), lambda i: (0, 0)),
                  pl.BlockSpec((c_out, c_mid), lambda i: (0, 0)),
                  pl.BlockSpec((1, c_out), lambda i: (0, 0))],
        out_specs=pl.BlockSpec((bn, c_out), lambda i: (i, 0)),
        out_shape=jax.ShapeDtypeStruct((rows, c_out), jnp.float32),
    )(g2d, w2, b2.reshape(1, c_mid), w3, b3.reshape(1, c_out))


def kernel(x, edge, fc1_Wl, fc1_Wr, fc1_att, fc1_bias, fc1_bl, fc1_br,
           fc3_W, fc3_b, gc1_Wl, gc1_Wr, gc1_att, gc1_bias, gc1_bl, gc1_br,
           gc2_Wl, gc2_Wr, gc2_att, gc2_bias, gc2_bl, gc2_br,
           gru0_Whh, gru0_Wih, gru0_bhh, gru0_bih,
           gru1_Whh, gru1_Wih, gru1_bhh, gru1_bih,
           rc1_Wl, rc1_Wr, rc1_att, rc1_bias, rc1_bl, rc1_br,
           rc2_W, rc2_b, rc3_W, rc3_b):
    t_, n, _ = x.shape
    mask_t = jnp.swapaxes(edge, 1, 2).astype(jnp.int8)

    out1 = _gat(x, mask_t, gc1_Wl, gc1_bl, gc1_Wr, gc1_br, gc1_att, gc1_bias)
    out2 = _gat(out1, mask_t, gc2_Wl, gc2_bl, gc2_Wr, gc2_br, gc2_att,
                gc2_bias)
    e = _gru2(out2, gru0_Wih, gru0_Whh, gru0_bih, gru0_bhh,
              gru1_Wih, gru1_Whh, gru1_bih, gru1_bhh)

    r_g = _gat(e[1:], mask_t[1:], rc1_Wl, rc1_bl, rc1_Wr, rc1_br, rc1_att,
               rc1_bias)
    f_g = _gat(e[:t_ - 1], mask_t[:t_ - 1], fc1_Wl, fc1_bl, fc1_Wr, fc1_br,
               fc1_att, fc1_bias)

    c_in = r_g.shape[-1]
    recon = _head(r_g.reshape(-1, c_in), rc2_W, rc2_b, rc3_W, rc3_b)
    fore = _head(f_g.reshape(-1, c_in), rc2_W, rc2_b, fc3_W, fc3_b)
    orig = rc3_W.shape[0]
    return (recon.reshape(t_ - 1, n, orig), fore.reshape(t_ - 1, n, orig), e)


# speedup vs baseline: 995.6718x; 5.5033x over previous
import functools

import jax
import jax.numpy as jnp
from jax.experimental import pallas as pl
from jax.experimental.pallas import tpu as pltpu

_HEADS = 2
_NEG = -1e30


def _dot(a, b, dims):
    return jax.lax.dot_general(
        a, b, (dims, ((), ())),
        preferred_element_type=jnp.float32,
        precision=jax.lax.Precision.HIGHEST)


def _gat_body(x_full_ref, x_blk_ref, m_ref, wl_ref, bl_ref, blc_ref, wr_ref,
              br_ref, att2_ref, att4_ref, bias_ref, o_ref, *, n, c, bi):
    hc = _HEADS * c
    xt = x_full_ref[0]
    xb = x_blk_ref[0]
    xl = _dot(xt, wl_ref[...], ((1,), (1,))) + bl_ref[...]
    xlt = _dot(wl_ref[...], xt, ((1,), (1,))) + blc_ref[...]
    xr = _dot(xb, wr_ref[...], ((1,), (1,))) + br_ref[...]

    att2 = att2_ref[...]
    rr = _dot(xr, att2, ((1,), (0,)))
    ll = _dot(att2, xlt, ((0,), (0,)))

    ib = pl.program_id(1)
    col = jax.lax.broadcasted_iota(jnp.int32, (bi, n), 1)
    row = jax.lax.broadcasted_iota(jnp.int32, (bi, n), 0) + ib * bi
    mf = m_ref[0].astype(jnp.float32)
    diag = jnp.where(col == row, 1.0, 0.0)
    neg = jnp.where(mf + diag > 0, 0.0, _NEG)

    accs = []
    for h in range(_HEADS):
        acc = jnp.zeros((bi, n), jnp.float32)
        for cc in range(h * c, (h + 1) * c):
            v = xr[:, cc:cc + 1] + xlt[cc:cc + 1, :]
            acc = acc + jnp.abs(v) * att4_ref[0, cc]
        accs.append(acc)

    out = None
    for h in range(_HEADS):
        alpha = (accs[h] + 0.6 * (rr[:, h:h + 1] + ll[h:h + 1, :])) + neg
        amax = jnp.max(alpha, axis=1, keepdims=True)
        ex = jnp.exp(alpha - amax)
        den = jnp.sum(ex, axis=1, keepdims=True) + 1e-16
        a = ex / den
        outh = _dot(a, xl[:, h * c:(h + 1) * c], ((1,), (0,)))
        out = outh if out is None else out + outh
    res = out * (1.0 / _HEADS) + bias_ref[...]
    o_ref[0] = jnp.where(res > 0, res, jnp.exp(jnp.minimum(res, 0.0)) - 1.0)


def _gat(x, mask_t, wl, bl, wr, br, att, bias):
    t_, n, in_ch = x.shape
    hc = wl.shape[0]
    c = hc // _HEADS
    bi = 128
    att_flat = att.reshape(hc)
    att2 = jnp.zeros((hc, _HEADS), jnp.float32)
    for h in range(_HEADS):
        att2 = att2.at[h * c:(h + 1) * c, h].set(att_flat[h * c:(h + 1) * c])
    body = functools.partial(_gat_body, n=n, c=c, bi=bi)
    return pl.pallas_call(
        body,
        grid=(t_, n // bi),
        in_specs=[
            pl.BlockSpec((1, n, in_ch), lambda t, i: (t, 0, 0)),
            pl.BlockSpec((1, bi, in_ch), lambda t, i: (t, i, 0)),
            pl.BlockSpec((1, bi, n), lambda t, i: (t, i, 0)),
            pl.BlockSpec((hc, in_ch), lambda t, i: (0, 0)),
            pl.BlockSpec((1, hc), lambda t, i: (0, 0)),
            pl.BlockSpec((hc, 1), lambda t, i: (0, 0)),
            pl.BlockSpec((hc, in_ch), lambda t, i: (0, 0)),
            pl.BlockSpec((1, hc), lambda t, i: (0, 0)),
            pl.BlockSpec((hc, _HEADS), lambda t, i: (0, 0)),
            pl.BlockSpec(memory_space=pltpu.SMEM),
            pl.BlockSpec((1, c), lambda t, i: (0, 0)),
        ],
        out_specs=pl.BlockSpec((1, bi, c), lambda t, i: (t, i, 0)),
        out_shape=jax.ShapeDtypeStruct((t_, n, c), jnp.float32),
    )(x, x, mask_t, wl, bl.reshape(1, hc), bl.reshape(hc, 1), wr,
      br.reshape(1, hc), att2, (0.4 * att_flat).reshape(1, hc),
      bias.reshape(1, c))


def _gru_body(e_ref, wih0_ref, whh0_ref, bih0_ref, bhh0_ref,
              wih1_ref, whh1_ref, bih1_ref, bhh1_ref, o_ref, *, t_, bn, hd):
    def step(xt, h, wih, whh, bih, bhh):
        gi = _dot(xt, wih, ((1,), (1,))) + bih
        gh = _dot(h, whh, ((1,), (1,))) + bhh
        r = 1.0 / (1.0 + jnp.exp(-(gi[:, :hd] + gh[:, :hd])))
        z = 1.0 / (1.0 + jnp.exp(-(gi[:, hd:2 * hd] + gh[:, hd:2 * hd])))
        nc = jnp.tanh(gi[:, 2 * hd:] + r * gh[:, 2 * hd:])
        return (1.0 - z) * nc + z * h

    h = jnp.zeros((bn, hd), jnp.float32)
    ys = []
    for t in range(t_):
        h = step(e_ref[t], h, wih0_ref[...], whh0_ref[...],
                 bih0_ref[...], bhh0_ref[...])
        ys.append(h)
    h = jnp.zeros((bn, hd), jnp.float32)
    for t in range(t_):
        h = step(ys[t], h, wih1_ref[...], whh1_ref[...],
                 bih1_ref[...], bhh1_ref[...])
        o_ref[t] = h


def _gru2(e, wih0, whh0, bih0, bhh0, wih1, whh1, bih1, bhh1):
    t_, n, hd = e.shape
    g3 = 3 * hd
    bn = 256
    body = functools.partial(_gru_body, t_=t_, bn=bn, hd=hd)
    wspec = pl.BlockSpec((g3, hd), lambda i: (0, 0))
    bspec = pl.BlockSpec((1, g3), lambda i: (0, 0))
    return pl.pallas_call(
        body,
        grid=(n // bn,),
        in_specs=[pl.BlockSpec((t_, bn, hd), lambda i: (0, i, 0)),
                  wspec, wspec, bspec, bspec, wspec, wspec, bspec, bspec],
        out_specs=pl.BlockSpec((t_, bn, hd), lambda i: (0, i, 0)),
        out_shape=jax.ShapeDtypeStruct((t_, n, hd), jnp.float32),
    )(e, wih0, whh0, bih0.reshape(1, g3), bhh0.reshape(1, g3),
      wih1, whh1, bih1.reshape(1, g3), bhh1.reshape(1, g3))


def _head_body(g_ref, w2_ref, b2_ref, w3_ref, b3_ref, o_ref):
    r = jnp.tanh(_dot(g_ref[...], w2_ref[...], ((1,), (1,))) + b2_ref[...])
    o_ref[...] = _dot(r, w3_ref[...], ((1,), (1,))) + b3_ref[...]


def _head(g2d, w2, b2, w3, b3):
    rows, c_in = g2d.shape
    c_mid = w2.shape[0]
    c_out = w3.shape[0]
    bn = 256
    return pl.pallas_call(
        _head_body,
        grid=(rows // bn,),
        in_specs=[pl.BlockSpec((bn, c_in), lambda i: (i, 0)),
                  pl.BlockSpec((c_mid, c_in), lambda i: (0, 0)),
                  pl.BlockSpec((1, c_mid), lambda i: (0, 0)),
                  pl.BlockSpec((c_out, c_mid), lambda i: (0, 0)),
                  pl.BlockSpec((1, c_out), lambda i: (0, 0))],
        out_specs=pl.BlockSpec((bn, c_out), lambda i: (i, 0)),
        out_shape=jax.ShapeDtypeStruct((rows, c_out), jnp.float32),
    )(g2d, w2, b2.reshape(1, c_mid), w3, b3.reshape(1, c_out))


def kernel(x, edge, fc1_Wl, fc1_Wr, fc1_att, fc1_bias, fc1_bl, fc1_br,
           fc3_W, fc3_b, gc1_Wl, gc1_Wr, gc1_att, gc1_bias, gc1_bl, gc1_br,
           gc2_Wl, gc2_Wr, gc2_att, gc2_bias, gc2_bl, gc2_br,
           gru0_Whh, gru0_Wih, gru0_bhh, gru0_bih,
           gru1_Whh, gru1_Wih, gru1_bhh, gru1_bih,
           rc1_Wl, rc1_Wr, rc1_att, rc1_bias, rc1_bl, rc1_br,
           rc2_W, rc2_b, rc3_W, rc3_b):
    t_, n, _ = x.shape
    mask_t = jnp.swapaxes(edge, 1, 2).astype(jnp.int8)

    out1 = _gat(x, mask_t, gc1_Wl, gc1_bl, gc1_Wr, gc1_br, gc1_att, gc1_bias)
    out2 = _gat(out1, mask_t, gc2_Wl, gc2_bl, gc2_Wr, gc2_br, gc2_att,
                gc2_bias)
    e = _gru2(out2, gru0_Wih, gru0_Whh, gru0_bih, gru0_bhh,
              gru1_Wih, gru1_Whh, gru1_bih, gru1_bhh)

    r_g = _gat(e[1:], mask_t[1:], rc1_Wl, rc1_bl, rc1_Wr, rc1_br, rc1_att,
               rc1_bias)
    f_g = _gat(e[:t_ - 1], mask_t[:t_ - 1], fc1_Wl, fc1_bl, fc1_Wr, fc1_br,
               fc1_att, fc1_bias)

    c_in = r_g.shape[-1]
    recon = _head(r_g.reshape(-1, c_in), rc2_W, rc2_b, rc3_W, rc3_b)
    fore = _head(f_g.reshape(-1, c_in), rc2_W, rc2_b, fc3_W, fc3_b)
    orig = rc3_W.shape[0]
    return (recon.reshape(t_ - 1, n, orig), fore.reshape(t_ - 1, n, orig), e)
